# baseline (speedup 1.0000x reference)
import functools

import jax
import jax.numpy as jnp
from jax import lax
from jax.experimental import pallas as pl
from jax.experimental.pallas import tpu as pltpu
from jax.experimental.pallas import tpu_sc as plsc

N = 10000
NP = 10240
E = 320000
H = 128
R = 1024
NBLK = NP // R

NC, NS = 2, 16
NT = NC * NS
EPT = E // NT
K = 80
NCHUNK = 128
EPC = NCHUNK * K
SL = NP // NS

@functools.cache
def _mesh():
    return plsc.VectorSubcoreMesh(core_axis_name="c", subcore_axis_name="s",
                                  num_cores=NC, num_subcores=NS)


def _deg_body(dst_hbm, deg_hbm, dbuf, hist, shared, tmp, accv):
    c = lax.axis_index("c")
    s = lax.axis_index("s")
    tile = c * NS + s
    base = tile * EPT

    def zero_hist(j, _):
        hist[pl.ds(j * 16, 16)] = jnp.zeros((16,), jnp.float32)
        return 0
    lax.fori_loop(0, NP // 16, zero_hist, 0)

    ones = jnp.ones((16,), jnp.float32)

    def outer(i, _):
        pltpu.sync_copy(dst_hbm.at[pl.ds(base + i * 2000, 2000)], dbuf)

        def inner(j, _):
            idx = dbuf[pl.ds(j * 16, 16)]
            plsc.addupdate_scatter(hist, [idx], ones)
            return 0
        lax.fori_loop(0, 2000 // 16, inner, 0)
        return 0
    lax.fori_loop(0, EPT // 2000, outer, 0)

    pltpu.sync_copy(hist, shared.at[s])
    plsc.subcore_barrier()

    def zero_acc(j, _):
        accv[pl.ds(j * 16, 16)] = jnp.zeros((16,), jnp.float32)
        return 0
    lax.fori_loop(0, SL // 16, zero_acc, 0)

    def red(t2, _):
        pltpu.sync_copy(shared.at[t2, pl.ds(s * SL, SL)], tmp)

        def add(j, _):
            accv[pl.ds(j * 16, 16)] = accv[pl.ds(j * 16, 16)] + tmp[pl.ds(j * 16, 16)]
            return 0
        lax.fori_loop(0, SL // 16, add, 0)
        return 0
    lax.fori_loop(0, NS, red, 0)

    pltpu.sync_copy(accv, deg_hbm.at[c, pl.ds(s * SL, SL)])


@functools.cache
def _deg_call():
    return pl.kernel(
        _deg_body,
        out_type=jax.ShapeDtypeStruct((NC, NP), jnp.float32),
        mesh=_mesh(),
        compiler_params=pltpu.CompilerParams(needs_layout_passes=False),
        scratch_types=[
            pltpu.VMEM((2000,), jnp.int32),
            pltpu.VMEM((NP,), jnp.float32),
            pltpu.VMEM_SHARED((NS, NP), jnp.float32),
            pltpu.VMEM((SL,), jnp.float32),
            pltpu.VMEM((SL,), jnp.float32),
        ],
    )


def _scatter_body(y_hbm, z_hbm, src_hbm, dst_hbm, parts_hbm, sidx, didx, rows, acc,
                  semi, semg, sems):
    c = lax.axis_index("c")
    s = lax.axis_index("s")
    tile = c * NS + s

    @pl.when(c == 0)
    def _():
        pltpu.sync_copy(y_hbm.at[pl.ds(s * SL, SL)], acc.at[pl.ds(s * SL, SL)])

    @pl.when(c == 1)
    def _():
        pltpu.sync_copy(z_hbm.at[pl.ds(s * SL, SL)], acc.at[pl.ds(s * SL, SL)])

    plsc.subcore_barrier()

    def idx_copy(j, jb):
        pltpu.async_copy(src_hbm.at[tile, j], sidx[jb], semi.at[jb])
        pltpu.async_copy(dst_hbm.at[tile, j], didx[jb], semi.at[jb])

    def idx_wait(j, jb):
        pltpu.make_async_copy(src_hbm.at[tile, j], sidx[jb], semi.at[jb]).wait()
        pltpu.make_async_copy(dst_hbm.at[tile, j], didx[jb], semi.at[jb]).wait()

    def gather(jb, b):
        pltpu.async_copy(y_hbm.at[sidx[jb]], rows.at[b], semg.at[b])

    def gather_wait(jb, b):
        pltpu.make_async_copy(y_hbm.at[sidx[jb]], rows.at[b], semg.at[b]).wait()

    def scatter(jb, b):
        pltpu.async_copy(rows.at[b], acc.at[didx[jb]], sems.at[b], add=True)

    def scatter_wait(jb, b):
        pltpu.make_async_copy(rows.at[b], acc.at[didx[jb]], sems.at[b]).wait()

    def body(j, u, first2, do_pre, do_post):
        b = u % 4
        jb = u % 8
        gather_wait(jb, b)
        scatter(jb, b)
        if not first2:
            scatter_wait((jb + 6) % 8, (b + 2) % 4)
        if do_pre:
            idx_copy(j + 6, (jb + 6) % 8)
        if do_post:
            idx_wait(j + 2, (jb + 2) % 8)
            gather((jb + 2) % 8, (b + 2) % 4)

    for m in range(6):
        idx_copy(m, m)
    for m in range(2):
        idx_wait(m, m)
        gather(m, m)
    for j in range(2):
        body(j, j, True, True, True)

    def grp(gg, _):
        for v in range(8):
            body(2 + gg * 8 + v, (2 + v) % 8, False, True, True)
        return 0
    lax.fori_loop(0, (NCHUNK - 8) // 8, grp, 0)
    for v in range(6):
        j = NCHUNK - 6 + v
        body(j, j % 8, False, False, v < 4)
    scatter_wait((NCHUNK - 2) % 8, (NCHUNK - 2) % 4)
    scatter_wait((NCHUNK - 1) % 8, (NCHUNK - 1) % 4)
    plsc.subcore_barrier()

    pltpu.sync_copy(acc.at[pl.ds(s * SL, SL)], parts_hbm.at[c, pl.ds(s * SL, SL)])


@functools.cache
def _scatter_call():
    return pl.kernel(
        _scatter_body,
        out_type=jax.ShapeDtypeStruct((NC, NP, H), jnp.float32),
        mesh=_mesh(),
        compiler_params=pltpu.CompilerParams(needs_layout_passes=False),
        scratch_types=[
            [pltpu.VMEM((K,), jnp.int32) for _ in range(8)],
            [pltpu.VMEM((K,), jnp.int32) for _ in range(8)],
            pltpu.VMEM((4, K, H), jnp.float32),
            pltpu.VMEM_SHARED((NP, H), jnp.float32),
            pltpu.SemaphoreType.DMA((8,)),
            pltpu.SemaphoreType.DMA((4,)),
            pltpu.SemaphoreType.DMA((4,)),
        ],
    )


def _dinv(d0_ref, d1_ref):
    return lax.rsqrt(d0_ref[...] + d1_ref[...] + 1.0)


def _affine_coefs(sums, gnw, gnb, ms):
    mu = sums[0:1, :] * (1.0 / N)
    ex2 = sums[1:2, :] * (1.0 / N)
    var = ex2 - mu * mu * ms * (2.0 - ms)
    inv = lax.rsqrt(var + 1e-5)
    a = gnw * inv
    cc = gnb - a * ms * mu
    return a, cc


def _a_first_body(x_ref, d0_ref, d1_ref, w_ref, y_ref):
    z = x_ref[...] * _dinv(d0_ref, d1_ref)
    y_ref[...] = jnp.dot(z, w_ref[...], preferred_element_type=jnp.float32)


def _a_norm_body(pre_ref, sums_ref, gnw_ref, gnb_ref, ms_ref, d0_ref, d1_ref,
                 w_ref, y_ref):
    a, cc = _affine_coefs(sums_ref[...], gnw_ref[...], gnb_ref[...], ms_ref[...])
    h = pre_ref[...] * a + cc
    y_ref[...] = jnp.dot(h * _dinv(d0_ref, d1_ref), w_ref[...],
                         preferred_element_type=jnp.float32)


_row_spec = pl.BlockSpec((R, H), lambda g: (g, 0))
_col_spec = pl.BlockSpec((R, 1), lambda g: (g, 0))
_w_spec = pl.BlockSpec((H, H), lambda g: (0, 0))
_vec_spec = pl.BlockSpec((1, H), lambda g: (0, 0))
_sums_spec = pl.BlockSpec((8, H), lambda g: (0, 0))


def _a_first(x, d0, d1, W):
    return pl.pallas_call(
        _a_first_body,
        grid=(NBLK,),
        in_specs=[_row_spec, _col_spec, _col_spec, _w_spec],
        out_specs=_row_spec,
        out_shape=jax.ShapeDtypeStruct((NP, H), jnp.float32),
    )(x, d0, d1, W)


def _a_norm(pre, sums, gnw, gnb, ms, d0, d1, W):
    return pl.pallas_call(
        _a_norm_body,
        grid=(NBLK,),
        in_specs=[_row_spec, _sums_spec, _vec_spec, _vec_spec, _vec_spec,
                  _col_spec, _col_spec, _w_spec],
        out_specs=_row_spec,
        out_shape=jax.ShapeDtypeStruct((NP, H), jnp.float32),
    )(pre, sums, gnw, gnb, ms, d0, d1, W)


def _b_body(p0_ref, p1_ref, d0_ref, d1_ref, b_ref, pre_ref, sums_ref):
    g = pl.program_id(0)
    pre = (p0_ref[...] + p1_ref[...]) * _dinv(d0_ref, d1_ref) + b_ref[...]
    row = g * R + lax.broadcasted_iota(jnp.int32, (R, H), 0)
    pre = jnp.where(row < N, pre, 0.0)
    pre_ref[...] = pre
    s1 = jnp.sum(pre, axis=0, keepdims=True)
    s2 = jnp.sum(pre * pre, axis=0, keepdims=True)
    blk = jnp.concatenate([s1, s2, jnp.zeros((6, H), jnp.float32)], axis=0)

    @pl.when(g == 0)
    def _():
        sums_ref[...] = blk

    @pl.when(g > 0)
    def _():
        sums_ref[...] = sums_ref[...] + blk


def _b_call(p0, p1, d0, d1, b):
    return pl.pallas_call(
        _b_body,
        grid=(NBLK,),
        in_specs=[_row_spec, _row_spec, _col_spec, _col_spec, _vec_spec],
        out_specs=[_row_spec, _sums_spec],
        out_shape=[
            jax.ShapeDtypeStruct((NP, H), jnp.float32),
            jax.ShapeDtypeStruct((8, H), jnp.float32),
        ],
    )(p0, p1, d0, d1, b)


def _f_body(pre_ref, sums_ref, gnw_ref, gnb_ref, ms_ref, out_ref):
    a, cc = _affine_coefs(sums_ref[...], gnw_ref[...], gnb_ref[...], ms_ref[...])
    out_ref[...] = pre_ref[...] * a + cc


def _f_call(pre, sums, gnw, gnb, ms):
    return pl.pallas_call(
        _f_body,
        grid=(NBLK,),
        in_specs=[_row_spec, _sums_spec, _vec_spec, _vec_spec, _vec_spec],
        out_specs=_row_spec,
        out_shape=jax.ShapeDtypeStruct((NP, H), jnp.float32),
    )(pre, sums, gnw, gnb, ms)


def kernel(x, edge_index, W0, b0, gn_w0, gn_b0, gn_ms0,
           W1, b1, gn_w1, gn_b1, gn_ms1,
           W2, b2, gn_w2, gn_b2, gn_ms2):
    src = edge_index[0].astype(jnp.int32)
    dst = edge_index[1].astype(jnp.int32)
    fill = jnp.broadcast_to(
        N + (jnp.arange(EPC - EPT, dtype=jnp.int32) % (NP - N)), (NT, EPC - EPT))
    src3 = jnp.concatenate([src.reshape(NT, EPT), fill], axis=1).reshape(NT, NCHUNK, K)
    dst3 = jnp.concatenate([dst.reshape(NT, EPT), fill], axis=1).reshape(NT, NCHUNK, K)
    xp = jnp.pad(x, ((0, NP - N), (0, 0)))
    zb = jnp.zeros((NP, H), jnp.float32)

    deg_parts = _deg_call()(dst)
    d0 = deg_parts[0].reshape(NP, 1)
    d1 = deg_parts[1].reshape(NP, 1)

    Ws = [W0, W1, W2]
    bs = [b0.reshape(1, H), b1.reshape(1, H), b2.reshape(1, H)]
    gws = [gn_w0.reshape(1, H), gn_w1.reshape(1, H), gn_w2.reshape(1, H)]
    gbs = [gn_b0.reshape(1, H), gn_b1.reshape(1, H), gn_b2.reshape(1, H)]
    gms = [gn_ms0.reshape(1, H), gn_ms1.reshape(1, H), gn_ms2.reshape(1, H)]

    pre, sums = None, None
    for i in range(3):
        if i == 0:
            y = _a_first(xp, d0, d1, Ws[i])
        else:
            y = _a_norm(pre, sums, gws[i - 1], gbs[i - 1], gms[i - 1], d0, d1, Ws[i])
        parts = _scatter_call()(y, zb, src3, dst3)
        pre, sums = _b_call(parts[0], parts[1], d0, d1, bs[i])

    out = _f_call(pre, sums, gws[2], gbs[2], gms[2])
    return out[:N]


# speedup vs baseline: 1.0013x; 1.0013x over previous
import functools

import jax
import jax.numpy as jnp
from jax import lax
from jax.experimental import pallas as pl
from jax.experimental.pallas import tpu as pltpu
from jax.experimental.pallas import tpu_sc as plsc

N = 10000
NP = 10240
E = 320000
H = 128
R = 1024
NBLK = NP // R

NC, NS = 2, 16
NT = NC * NS
EPT = E // NT
K = 80
NCHUNK = 128
EPC = NCHUNK * K
SL = NP // NS

@functools.cache
def _mesh():
    return plsc.VectorSubcoreMesh(core_axis_name="c", subcore_axis_name="s",
                                  num_cores=NC, num_subcores=NS)


def _deg_body(dst_hbm, deg_hbm, dbuf, hist, shared, tmp, accv):
    c = lax.axis_index("c")
    s = lax.axis_index("s")
    tile = c * NS + s
    base = tile * EPT

    def zero_hist(j, _):
        hist[pl.ds(j * 16, 16)] = jnp.zeros((16,), jnp.float32)
        return 0
    lax.fori_loop(0, NP // 16, zero_hist, 0)

    ones = jnp.ones((16,), jnp.float32)

    def outer(i, _):
        pltpu.sync_copy(dst_hbm.at[pl.ds(base + i * 2000, 2000)], dbuf)

        def inner(j, _):
            idx = dbuf[pl.ds(j * 16, 16)]
            plsc.addupdate_scatter(hist, [idx], ones)
            return 0
        lax.fori_loop(0, 2000 // 16, inner, 0)
        return 0
    lax.fori_loop(0, EPT // 2000, outer, 0)

    pltpu.sync_copy(hist, shared.at[s])
    plsc.subcore_barrier()

    def zero_acc(j, _):
        accv[pl.ds(j * 16, 16)] = jnp.zeros((16,), jnp.float32)
        return 0
    lax.fori_loop(0, SL // 16, zero_acc, 0)

    def red(t2, _):
        pltpu.sync_copy(shared.at[t2, pl.ds(s * SL, SL)], tmp)

        def add(j, _):
            accv[pl.ds(j * 16, 16)] = accv[pl.ds(j * 16, 16)] + tmp[pl.ds(j * 16, 16)]
            return 0
        lax.fori_loop(0, SL // 16, add, 0)
        return 0
    lax.fori_loop(0, NS, red, 0)

    pltpu.sync_copy(accv, deg_hbm.at[c, pl.ds(s * SL, SL)])


@functools.cache
def _deg_call():
    return pl.kernel(
        _deg_body,
        out_type=jax.ShapeDtypeStruct((NC, NP), jnp.float32),
        mesh=_mesh(),
        compiler_params=pltpu.CompilerParams(needs_layout_passes=False),
        scratch_types=[
            pltpu.VMEM((2000,), jnp.int32),
            pltpu.VMEM((NP,), jnp.float32),
            pltpu.VMEM_SHARED((NS, NP), jnp.float32),
            pltpu.VMEM((SL,), jnp.float32),
            pltpu.VMEM((SL,), jnp.float32),
        ],
    )


def _scatter_body(y_hbm, src_hbm, dst_hbm, parts_hbm, sidx, didx, rows, zbuf, acc,
                  semi, semg, sems):
    c = lax.axis_index("c")
    s = lax.axis_index("s")
    tile = c * NS + s

    @pl.when(c == 0)
    def _():
        pltpu.sync_copy(y_hbm.at[pl.ds(s * SL, SL)], acc.at[pl.ds(s * SL, SL)])

    @pl.when(c == 1)
    def _():
        def zr(i, _):
            def zc(l, _):
                zbuf[i, pl.ds(l * 16, 16)] = jnp.zeros((16,), jnp.float32)
                return 0
            lax.fori_loop(0, H // 16, zc, 0)
            return 0
        lax.fori_loop(0, 16, zr, 0)

        def zcopy(i, _):
            pltpu.sync_copy(zbuf, acc.at[pl.ds(s * SL + i * 16, 16)])
            return 0
        lax.fori_loop(0, SL // 16, zcopy, 0)

    plsc.subcore_barrier()

    def idx_copy(j, jb):
        pltpu.async_copy(src_hbm.at[tile, j], sidx[jb], semi.at[jb])
        pltpu.async_copy(dst_hbm.at[tile, j], didx[jb], semi.at[jb])

    def idx_wait(j, jb):
        pltpu.make_async_copy(src_hbm.at[tile, j], sidx[jb], semi.at[jb]).wait()
        pltpu.make_async_copy(dst_hbm.at[tile, j], didx[jb], semi.at[jb]).wait()

    def gather(jb, b):
        pltpu.async_copy(y_hbm.at[sidx[jb]], rows.at[b], semg.at[b])

    def gather_wait(jb, b):
        pltpu.make_async_copy(y_hbm.at[sidx[jb]], rows.at[b], semg.at[b]).wait()

    def scatter(jb, b):
        pltpu.async_copy(rows.at[b], acc.at[didx[jb]], sems.at[b], add=True)

    def scatter_wait(jb, b):
        pltpu.make_async_copy(rows.at[b], acc.at[didx[jb]], sems.at[b]).wait()

    def body(j, u, first2, do_pre, do_post):
        b = u % 4
        jb = u % 8
        gather_wait(jb, b)
        scatter(jb, b)
        if not first2:
            scatter_wait((jb + 6) % 8, (b + 2) % 4)
        if do_pre:
            idx_copy(j + 6, (jb + 6) % 8)
        if do_post:
            idx_wait(j + 2, (jb + 2) % 8)
            gather((jb + 2) % 8, (b + 2) % 4)

    for m in range(6):
        idx_copy(m, m)
    for m in range(2):
        idx_wait(m, m)
        gather(m, m)
    for j in range(2):
        body(j, j, True, True, True)

    def grp(gg, _):
        for v in range(8):
            body(2 + gg * 8 + v, (2 + v) % 8, False, True, True)
        return 0
    lax.fori_loop(0, (NCHUNK - 8) // 8, grp, 0)
    for v in range(6):
        j = NCHUNK - 6 + v
        body(j, j % 8, False, False, v < 4)
    scatter_wait((NCHUNK - 2) % 8, (NCHUNK - 2) % 4)
    scatter_wait((NCHUNK - 1) % 8, (NCHUNK - 1) % 4)
    plsc.subcore_barrier()

    pltpu.sync_copy(acc.at[pl.ds(s * SL, SL)], parts_hbm.at[c, pl.ds(s * SL, SL)])


@functools.cache
def _scatter_call():
    return pl.kernel(
        _scatter_body,
        out_type=jax.ShapeDtypeStruct((NC, NP, H), jnp.float32),
        mesh=_mesh(),
        compiler_params=pltpu.CompilerParams(needs_layout_passes=False),
        scratch_types=[
            [pltpu.VMEM((K,), jnp.int32) for _ in range(8)],
            [pltpu.VMEM((K,), jnp.int32) for _ in range(8)],
            pltpu.VMEM((4, K, H), jnp.float32),
            pltpu.VMEM((16, H), jnp.float32),
            pltpu.VMEM_SHARED((NP, H), jnp.float32),
            pltpu.SemaphoreType.DMA((8,)),
            pltpu.SemaphoreType.DMA((4,)),
            pltpu.SemaphoreType.DMA((4,)),
        ],
    )


def _dinv(d0_ref, d1_ref):
    return lax.rsqrt(d0_ref[...] + d1_ref[...] + 1.0)


def _affine_coefs(sums, gnw, gnb, ms):
    mu = sums[0:1, :] * (1.0 / N)
    ex2 = sums[1:2, :] * (1.0 / N)
    var = ex2 - mu * mu * ms * (2.0 - ms)
    inv = lax.rsqrt(var + 1e-5)
    a = gnw * inv
    cc = gnb - a * ms * mu
    return a, cc


def _a_first_body(x_ref, d0_ref, d1_ref, w_ref, y_ref):
    z = x_ref[...] * _dinv(d0_ref, d1_ref)
    y_ref[...] = jnp.dot(z, w_ref[...], preferred_element_type=jnp.float32)


def _a_norm_body(pre_ref, sums_ref, gnw_ref, gnb_ref, ms_ref, d0_ref, d1_ref,
                 w_ref, y_ref):
    a, cc = _affine_coefs(sums_ref[...], gnw_ref[...], gnb_ref[...], ms_ref[...])
    h = pre_ref[...] * a + cc
    y_ref[...] = jnp.dot(h * _dinv(d0_ref, d1_ref), w_ref[...],
                         preferred_element_type=jnp.float32)


_row_spec = pl.BlockSpec((R, H), lambda g: (g, 0))
_col_spec = pl.BlockSpec((R, 1), lambda g: (g, 0))
_w_spec = pl.BlockSpec((H, H), lambda g: (0, 0))
_vec_spec = pl.BlockSpec((1, H), lambda g: (0, 0))
_sums_spec = pl.BlockSpec((8, H), lambda g: (0, 0))


def _a_first(x, d0, d1, W):
    return pl.pallas_call(
        _a_first_body,
        grid=(NBLK,),
        in_specs=[_row_spec, _col_spec, _col_spec, _w_spec],
        out_specs=_row_spec,
        out_shape=jax.ShapeDtypeStruct((NP, H), jnp.float32),
    )(x, d0, d1, W)


def _a_norm(pre, sums, gnw, gnb, ms, d0, d1, W):
    return pl.pallas_call(
        _a_norm_body,
        grid=(NBLK,),
        in_specs=[_row_spec, _sums_spec, _vec_spec, _vec_spec, _vec_spec,
                  _col_spec, _col_spec, _w_spec],
        out_specs=_row_spec,
        out_shape=jax.ShapeDtypeStruct((NP, H), jnp.float32),
    )(pre, sums, gnw, gnb, ms, d0, d1, W)


def _b_body(p0_ref, p1_ref, d0_ref, d1_ref, b_ref, pre_ref, sums_ref):
    g = pl.program_id(0)
    pre = (p0_ref[...] + p1_ref[...]) * _dinv(d0_ref, d1_ref) + b_ref[...]
    row = g * R + lax.broadcasted_iota(jnp.int32, (R, H), 0)
    pre = jnp.where(row < N, pre, 0.0)
    pre_ref[...] = pre
    s1 = jnp.sum(pre, axis=0, keepdims=True)
    s2 = jnp.sum(pre * pre, axis=0, keepdims=True)
    blk = jnp.concatenate([s1, s2, jnp.zeros((6, H), jnp.float32)], axis=0)

    @pl.when(g == 0)
    def _():
        sums_ref[...] = blk

    @pl.when(g > 0)
    def _():
        sums_ref[...] = sums_ref[...] + blk


def _b_call(p0, p1, d0, d1, b):
    return pl.pallas_call(
        _b_body,
        grid=(NBLK,),
        in_specs=[_row_spec, _row_spec, _col_spec, _col_spec, _vec_spec],
        out_specs=[_row_spec, _sums_spec],
        out_shape=[
            jax.ShapeDtypeStruct((NP, H), jnp.float32),
            jax.ShapeDtypeStruct((8, H), jnp.float32),
        ],
    )(p0, p1, d0, d1, b)


def _f_body(pre_ref, sums_ref, gnw_ref, gnb_ref, ms_ref, out_ref):
    a, cc = _affine_coefs(sums_ref[...], gnw_ref[...], gnb_ref[...], ms_ref[...])
    out_ref[...] = pre_ref[...] * a + cc


def _f_call(pre, sums, gnw, gnb, ms):
    return pl.pallas_call(
        _f_body,
        grid=(NBLK,),
        in_specs=[_row_spec, _sums_spec, _vec_spec, _vec_spec, _vec_spec],
        out_specs=_row_spec,
        out_shape=jax.ShapeDtypeStruct((NP, H), jnp.float32),
    )(pre, sums, gnw, gnb, ms)


def kernel(x, edge_index, W0, b0, gn_w0, gn_b0, gn_ms0,
           W1, b1, gn_w1, gn_b1, gn_ms1,
           W2, b2, gn_w2, gn_b2, gn_ms2):
    src = edge_index[0].astype(jnp.int32)
    dst = edge_index[1].astype(jnp.int32)
    fill = jnp.broadcast_to(
        N + (jnp.arange(EPC - EPT, dtype=jnp.int32) % (NP - N)), (NT, EPC - EPT))
    src3 = jnp.concatenate([src.reshape(NT, EPT), fill], axis=1).reshape(NT, NCHUNK, K)
    dst3 = jnp.concatenate([dst.reshape(NT, EPT), fill], axis=1).reshape(NT, NCHUNK, K)
    xp = jnp.pad(x, ((0, NP - N), (0, 0)))

    deg_parts = _deg_call()(dst)
    d0 = deg_parts[0].reshape(NP, 1)
    d1 = deg_parts[1].reshape(NP, 1)

    Ws = [W0, W1, W2]
    bs = [b0.reshape(1, H), b1.reshape(1, H), b2.reshape(1, H)]
    gws = [gn_w0.reshape(1, H), gn_w1.reshape(1, H), gn_w2.reshape(1, H)]
    gbs = [gn_b0.reshape(1, H), gn_b1.reshape(1, H), gn_b2.reshape(1, H)]
    gms = [gn_ms0.reshape(1, H), gn_ms1.reshape(1, H), gn_ms2.reshape(1, H)]

    pre, sums = None, None
    for i in range(3):
        if i == 0:
            y = _a_first(xp, d0, d1, Ws[i])
        else:
            y = _a_norm(pre, sums, gws[i - 1], gbs[i - 1], gms[i - 1], d0, d1, Ws[i])
        parts = _scatter_call()(y, src3, dst3)
        pre, sums = _b_call(parts[0], parts[1], d0, d1, bs[i])

    out = _f_call(pre, sums, gws[2], gbs[2], gms[2])
    return out[:N]


# speedup vs baseline: 1.0092x; 1.0078x over previous
import functools

import jax
import jax.numpy as jnp
from jax import lax
from jax.experimental import pallas as pl
from jax.experimental.pallas import tpu as pltpu
from jax.experimental.pallas import tpu_sc as plsc

N = 10000
NP = 10240
E = 320000
H = 128
R = 1024
NBLK = NP // R

NC, NS = 2, 16
NT = NC * NS
EPT = E // NT
K = 80
NCHUNK = 128
EPC = NCHUNK * K
SL = NP // NS

@functools.cache
def _mesh():
    return plsc.VectorSubcoreMesh(core_axis_name="c", subcore_axis_name="s",
                                  num_cores=NC, num_subcores=NS)


def _deg_body(dst_hbm, deg_hbm, dbuf, hist, shared, tmp, accv):
    c = lax.axis_index("c")
    s = lax.axis_index("s")
    tile = c * NS + s
    base = tile * EPT

    def zero_hist(j, _):
        hist[pl.ds(j * 16, 16)] = jnp.zeros((16,), jnp.float32)
        return 0
    lax.fori_loop(0, NP // 16, zero_hist, 0)

    ones = jnp.ones((16,), jnp.float32)

    def outer(i, _):
        pltpu.sync_copy(dst_hbm.at[pl.ds(base + i * 2000, 2000)], dbuf)

        def inner(j, _):
            idx = dbuf[pl.ds(j * 16, 16)]
            plsc.addupdate_scatter(hist, [idx], ones)
            return 0
        lax.fori_loop(0, 2000 // 16, inner, 0)
        return 0
    lax.fori_loop(0, EPT // 2000, outer, 0)

    pltpu.sync_copy(hist, shared.at[s])
    plsc.subcore_barrier()

    def zero_acc(j, _):
        accv[pl.ds(j * 16, 16)] = jnp.zeros((16,), jnp.float32)
        return 0
    lax.fori_loop(0, SL // 16, zero_acc, 0)

    def red(t2, _):
        pltpu.sync_copy(shared.at[t2, pl.ds(s * SL, SL)], tmp)

        def add(j, _):
            accv[pl.ds(j * 16, 16)] = accv[pl.ds(j * 16, 16)] + tmp[pl.ds(j * 16, 16)]
            return 0
        lax.fori_loop(0, SL // 16, add, 0)
        return 0
    lax.fori_loop(0, NS, red, 0)

    pltpu.sync_copy(accv, deg_hbm.at[c, pl.ds(s * SL, SL)])


@functools.cache
def _deg_call():
    return pl.kernel(
        _deg_body,
        out_type=jax.ShapeDtypeStruct((NC, NP), jnp.float32),
        mesh=_mesh(),
        compiler_params=pltpu.CompilerParams(needs_layout_passes=False),
        scratch_types=[
            pltpu.VMEM((2000,), jnp.int32),
            pltpu.VMEM((NP,), jnp.float32),
            pltpu.VMEM_SHARED((NS, NP), jnp.float32),
            pltpu.VMEM((SL,), jnp.float32),
            pltpu.VMEM((SL,), jnp.float32),
        ],
    )


def _scatter_body(y_hbm, src_hbm, dst_hbm, parts_hbm, sidx, didx, rows, zbuf, acc,
                  semi, semg, sems):
    c = lax.axis_index("c")
    s = lax.axis_index("s")
    tile = c * NS + s

    def zr(i, _):
        def zc(l, _):
            zbuf[i, pl.ds(l * 16, 16)] = jnp.zeros((16,), jnp.float32)
            return 0
        lax.fori_loop(0, H // 16, zc, 0)
        return 0
    lax.fori_loop(0, 16, zr, 0)

    def zcopy(i, _):
        pltpu.sync_copy(zbuf, acc.at[pl.ds(s * SL + i * 16, 16)])
        return 0
    lax.fori_loop(0, SL // 16, zcopy, 0)

    plsc.subcore_barrier()

    def idx_copy(j, jb):
        pltpu.async_copy(src_hbm.at[tile, j], sidx[jb], semi.at[jb])
        pltpu.async_copy(dst_hbm.at[tile, j], didx[jb], semi.at[jb])

    def idx_wait(j, jb):
        pltpu.make_async_copy(src_hbm.at[tile, j], sidx[jb], semi.at[jb]).wait()
        pltpu.make_async_copy(dst_hbm.at[tile, j], didx[jb], semi.at[jb]).wait()

    def gather(jb, b):
        pltpu.async_copy(y_hbm.at[sidx[jb]], rows.at[b], semg.at[b])

    def gather_wait(jb, b):
        pltpu.make_async_copy(y_hbm.at[sidx[jb]], rows.at[b], semg.at[b]).wait()

    def scatter(jb, b):
        pltpu.async_copy(rows.at[b], acc.at[didx[jb]], sems.at[b], add=True)

    def scatter_wait(jb, b):
        pltpu.make_async_copy(rows.at[b], acc.at[didx[jb]], sems.at[b]).wait()

    def body(j, u, first2, do_pre, do_post):
        b = u % 4
        jb = u % 8
        gather_wait(jb, b)
        scatter(jb, b)
        if not first2:
            scatter_wait((jb + 6) % 8, (b + 2) % 4)
        if do_pre:
            idx_copy(j + 6, (jb + 6) % 8)
        if do_post:
            idx_wait(j + 2, (jb + 2) % 8)
            gather((jb + 2) % 8, (b + 2) % 4)

    for m in range(6):
        idx_copy(m, m)
    for m in range(2):
        idx_wait(m, m)
        gather(m, m)
    for j in range(2):
        body(j, j, True, True, True)

    def grp(gg, _):
        for v in range(8):
            body(2 + gg * 8 + v, (2 + v) % 8, False, True, True)
        return 0
    lax.fori_loop(0, (NCHUNK - 8) // 8, grp, 0)
    for v in range(6):
        j = NCHUNK - 6 + v
        body(j, j % 8, False, False, v < 4)
    scatter_wait((NCHUNK - 2) % 8, (NCHUNK - 2) % 4)
    scatter_wait((NCHUNK - 1) % 8, (NCHUNK - 1) % 4)
    plsc.subcore_barrier()

    pltpu.sync_copy(acc.at[pl.ds(s * SL, SL)], parts_hbm.at[c, pl.ds(s * SL, SL)])


@functools.cache
def _scatter_call():
    return pl.kernel(
        _scatter_body,
        out_type=jax.ShapeDtypeStruct((NC, NP, H), jnp.float32),
        mesh=_mesh(),
        compiler_params=pltpu.CompilerParams(needs_layout_passes=False),
        scratch_types=[
            [pltpu.VMEM((K,), jnp.int32) for _ in range(8)],
            [pltpu.VMEM((K,), jnp.int32) for _ in range(8)],
            pltpu.VMEM((4, K, H), jnp.float32),
            pltpu.VMEM((16, H), jnp.float32),
            pltpu.VMEM_SHARED((NP, H), jnp.float32),
            pltpu.SemaphoreType.DMA((8,)),
            pltpu.SemaphoreType.DMA((4,)),
            pltpu.SemaphoreType.DMA((4,)),
        ],
    )


def _dinv(d0_ref, d1_ref):
    return lax.rsqrt(d0_ref[...] + d1_ref[...] + 1.0)


def _affine_coefs(sums, gnw, gnb, ms):
    mu = sums[0:1, :] * (1.0 / N)
    ex2 = sums[1:2, :] * (1.0 / N)
    var = ex2 - mu * mu * ms * (2.0 - ms)
    inv = lax.rsqrt(var + 1e-5)
    a = gnw * inv
    cc = gnb - a * ms * mu
    return a, cc


def _a_first_body(x_ref, d0_ref, d1_ref, w_ref, y_ref):
    z = x_ref[...] * _dinv(d0_ref, d1_ref)
    y_ref[...] = jnp.dot(z, w_ref[...], preferred_element_type=jnp.float32)


def _a_norm_body(pre_ref, sums_ref, gnw_ref, gnb_ref, ms_ref, d0_ref, d1_ref,
                 w_ref, y_ref):
    a, cc = _affine_coefs(sums_ref[...], gnw_ref[...], gnb_ref[...], ms_ref[...])
    h = pre_ref[...] * a + cc
    y_ref[...] = jnp.dot(h * _dinv(d0_ref, d1_ref), w_ref[...],
                         preferred_element_type=jnp.float32)


_row_spec = pl.BlockSpec((R, H), lambda g: (g, 0))
_col_spec = pl.BlockSpec((R, 1), lambda g: (g, 0))
_w_spec = pl.BlockSpec((H, H), lambda g: (0, 0))
_vec_spec = pl.BlockSpec((1, H), lambda g: (0, 0))
_sums_spec = pl.BlockSpec((8, H), lambda g: (0, 0))


def _a_first(x, d0, d1, W):
    return pl.pallas_call(
        _a_first_body,
        grid=(NBLK,),
        in_specs=[_row_spec, _col_spec, _col_spec, _w_spec],
        out_specs=_row_spec,
        out_shape=jax.ShapeDtypeStruct((NP, H), jnp.float32),
    )(x, d0, d1, W)


def _a_norm(pre, sums, gnw, gnb, ms, d0, d1, W):
    return pl.pallas_call(
        _a_norm_body,
        grid=(NBLK,),
        in_specs=[_row_spec, _sums_spec, _vec_spec, _vec_spec, _vec_spec,
                  _col_spec, _col_spec, _w_spec],
        out_specs=_row_spec,
        out_shape=jax.ShapeDtypeStruct((NP, H), jnp.float32),
    )(pre, sums, gnw, gnb, ms, d0, d1, W)


def _b_body(p0_ref, p1_ref, y_ref, d0_ref, d1_ref, b_ref, pre_ref, sums_ref):
    g = pl.program_id(0)
    pre = (p0_ref[...] + p1_ref[...] + y_ref[...]) * _dinv(d0_ref, d1_ref) + b_ref[...]
    row = g * R + lax.broadcasted_iota(jnp.int32, (R, H), 0)
    pre = jnp.where(row < N, pre, 0.0)
    pre_ref[...] = pre
    s1 = jnp.sum(pre, axis=0, keepdims=True)
    s2 = jnp.sum(pre * pre, axis=0, keepdims=True)
    blk = jnp.concatenate([s1, s2, jnp.zeros((6, H), jnp.float32)], axis=0)

    @pl.when(g == 0)
    def _():
        sums_ref[...] = blk

    @pl.when(g > 0)
    def _():
        sums_ref[...] = sums_ref[...] + blk


def _b_call(p0, p1, y, d0, d1, b):
    return pl.pallas_call(
        _b_body,
        grid=(NBLK,),
        in_specs=[_row_spec, _row_spec, _row_spec, _col_spec, _col_spec, _vec_spec],
        out_specs=[_row_spec, _sums_spec],
        out_shape=[
            jax.ShapeDtypeStruct((NP, H), jnp.float32),
            jax.ShapeDtypeStruct((8, H), jnp.float32),
        ],
    )(p0, p1, y, d0, d1, b)


def _f_body(pre_ref, sums_ref, gnw_ref, gnb_ref, ms_ref, out_ref):
    a, cc = _affine_coefs(sums_ref[...], gnw_ref[...], gnb_ref[...], ms_ref[...])
    out_ref[...] = pre_ref[...] * a + cc


def _f_call(pre, sums, gnw, gnb, ms):
    return pl.pallas_call(
        _f_body,
        grid=(NBLK,),
        in_specs=[_row_spec, _sums_spec, _vec_spec, _vec_spec, _vec_spec],
        out_specs=_row_spec,
        out_shape=jax.ShapeDtypeStruct((NP, H), jnp.float32),
    )(pre, sums, gnw, gnb, ms)


def kernel(x, edge_index, W0, b0, gn_w0, gn_b0, gn_ms0,
           W1, b1, gn_w1, gn_b1, gn_ms1,
           W2, b2, gn_w2, gn_b2, gn_ms2):
    src = edge_index[0].astype(jnp.int32)
    dst = edge_index[1].astype(jnp.int32)
    fill = jnp.broadcast_to(
        N + (jnp.arange(EPC - EPT, dtype=jnp.int32) % (NP - N)), (NT, EPC - EPT))
    src3 = jnp.concatenate([src.reshape(NT, EPT), fill], axis=1).reshape(NT, NCHUNK, K)
    dst3 = jnp.concatenate([dst.reshape(NT, EPT), fill], axis=1).reshape(NT, NCHUNK, K)
    xp = jnp.pad(x, ((0, NP - N), (0, 0)))

    deg_parts = lax.optimization_barrier(_deg_call()(dst))
    d0 = deg_parts[0].reshape(NP, 1)
    d1 = deg_parts[1].reshape(NP, 1)

    Ws = [W0, W1, W2]
    bs = [b0.reshape(1, H), b1.reshape(1, H), b2.reshape(1, H)]
    gws = [gn_w0.reshape(1, H), gn_w1.reshape(1, H), gn_w2.reshape(1, H)]
    gbs = [gn_b0.reshape(1, H), gn_b1.reshape(1, H), gn_b2.reshape(1, H)]
    gms = [gn_ms0.reshape(1, H), gn_ms1.reshape(1, H), gn_ms2.reshape(1, H)]

    pre, sums = None, None
    for i in range(3):
        if i == 0:
            y = _a_first(xp, d0, d1, Ws[i])
        else:
            y = _a_norm(pre, sums, gws[i - 1], gbs[i - 1], gms[i - 1], d0, d1, Ws[i])
        y = lax.optimization_barrier(y)
        parts = _scatter_call()(y, src3, dst3)
        parts = lax.optimization_barrier(parts)
        pre, sums = _b_call(parts[0], parts[1], y, d0, d1, bs[i])

    out = _f_call(pre, sums, gws[2], gbs[2], gms[2])
    return out[:N]
